# baseline (speedup 1.0000x reference)
import dataclasses
import functools

import jax
import jax.numpy as jnp
import numpy as np
from jax import lax
from jax.experimental import pallas as pl
from jax.experimental.pallas import tpu as pltpu
from jax.experimental.pallas import tpu_sc as plsc

_MASK_RATIO = 0.5
_G = 4
_H, _W = 2048, 4096
_NGH, _NGW = _H // _G, _W // _G
_NG = _NGH * _NGW
_NMASK = int(_MASK_RATIO * _NG)

_BR = 256

_NC, _NS = 2, 16
_NWORK = _NC * _NS
_ROWS_PER_W = _NGH // _NWORK
_ELEMS_PER_W = _ROWS_PER_W * _W
_SC_UNROLL = 4


@functools.lru_cache(maxsize=1)
def _perm_np():
    try:
        cpu = jax.local_devices(backend="cpu")[0]
    except RuntimeError:
        cpu = None
    try:
        with jax.ensure_compile_time_eval():
            if cpu is not None:
                with jax.default_device(cpu):
                    p = jax.random.permutation(jax.random.key(42), _NG)
            else:
                p = jax.random.permutation(jax.random.key(42), _NG)
        return np.asarray(p)
    except AttributeError:
        return np.concatenate([np.arange(0, _NG, 2), np.arange(1, _NG, 2)])


@functools.lru_cache(maxsize=1)
def _scatter_offsets_np():
    masked = _perm_np()[:_NMASK]
    gh = masked // _NGW
    gw = masked % _NGW
    worker = gh // _ROWS_PER_W
    local = (gh % _ROWS_PER_W) * _W + _G * gw
    buckets = []
    for w in range(_NWORK):
        base = local[worker == w]
        offs = (base[:, None] + np.arange(_G)[None, :]).reshape(-1)
        buckets.append(offs)
    padn = max(len(b) for b in buckets)
    step = 16 * _SC_UNROLL
    padn = ((padn + step - 1) // step) * step
    out = np.zeros((_NWORK, padn), np.int32)
    for w, b in enumerate(buckets):
        pad_val = b[-1] if len(b) else 0
        out[w, : len(b)] = b
        out[w, len(b):] = pad_val
    return out


def _sc_mask_cols(idx):
    padn = idx.shape[1]
    mesh = plsc.VectorSubcoreMesh(core_axis_name="c", subcore_axis_name="s")
    cp = pltpu.CompilerParams()
    if "needs_layout_passes" in pltpu.CompilerParams.__dataclass_fields__:
        cp = dataclasses.replace(cp, needs_layout_passes=False)

    @functools.partial(
        pl.kernel,
        out_type=jax.ShapeDtypeStruct((_NGH * _W,), jnp.float32),
        mesh=mesh,
        compiler_params=cp,
        scratch_types=[
            pltpu.VMEM((padn,), jnp.int32),
            pltpu.VMEM((_ELEMS_PER_W,), jnp.float32),
        ],
    )
    def sc_build(idx_hbm, out_hbm, idx_v, buf_v):
        wid = lax.axis_index("s") * _NC + lax.axis_index("c")
        pltpu.sync_copy(idx_hbm.at[wid], idx_v)

        ones = jnp.full((16,), 1.0, jnp.float32)
        zeros = jnp.zeros((16,), jnp.float32)

        @pl.loop(0, _ELEMS_PER_W, step=16 * _SC_UNROLL)
        def _(i):
            for j in range(_SC_UNROLL):
                buf_v[pl.ds(i + 16 * j, 16)] = ones

        @pl.loop(0, padn, step=16 * _SC_UNROLL)
        def _(i):
            for j in range(_SC_UNROLL):
                iv = idx_v[pl.ds(i + 16 * j, 16)]
                plsc.store_scatter(buf_v, [iv], zeros)

        pltpu.sync_copy(buf_v, out_hbm.at[pl.ds(wid * _ELEMS_PER_W, _ELEMS_PER_W)])

    return sc_build(idx)


def _mul_body(x_ref, m_ref, o_ref):
    for k in range(_BR // _G):
        o_ref[_G * k:_G * (k + 1), :] = (
            x_ref[_G * k:_G * (k + 1), :] * m_ref[k:k + 1, :]
        )


def kernel(spectrogram):
    x = spectrogram.reshape(_H, _W)
    m = _sc_mask_cols(jnp.asarray(_scatter_offsets_np())).reshape(_NGH, _W)
    grid = (_H // _BR,)
    out = pl.pallas_call(
        _mul_body,
        grid=grid,
        in_specs=[
            pl.BlockSpec((_BR, _W), lambda i: (i, 0)),
            pl.BlockSpec((_BR // _G, _W), lambda i: (i, 0)),
        ],
        out_specs=pl.BlockSpec((_BR, _W), lambda i: (i, 0)),
        out_shape=jax.ShapeDtypeStruct((_H, _W), jnp.float32),
        compiler_params=pltpu.CompilerParams(
            dimension_semantics=("arbitrary",),
        ),
    )(x, m)
    return out.reshape(1, _H, _W)
